# baseline (speedup 1.0000x reference)
import jax
import jax.numpy as jnp
from jax.experimental import pallas as pl
from jax.experimental.pallas import tpu as pltpu

_PNUM = 1024
_RB = 8
_NO = _PNUM // 128
_UNROLL = 16


def _poly_loss_kernel(p_ref, g_ref, o_ref):
    px = p_ref[0, 0:1, :]
    py = p_ref[0, 1:2, :]
    gx = g_ref[0, 0:1, :]
    gy = g_ref[0, 1:2, :]

    def _roll(v, r):
        return v if r == 0 else jnp.roll(v, -r, axis=1)

    g16 = jnp.concatenate(
        [_roll(gx, r) for r in range(_RB)] + [_roll(gy, r) for r in range(_RB)],
        axis=0,
    )

    p16 = jnp.concatenate(
        [jnp.broadcast_to(px, (_RB, _PNUM)), jnp.broadcast_to(py, (_RB, _PNUM))],
        axis=0,
    )
    po = [p16] + [jnp.roll(p16, 128 * o, axis=1) for o in range(1, _NO)]

    def smooth2(d):
        a = jnp.abs(d)
        m = jnp.minimum(a, 1.0)
        return m * (a + a - m)

    def body(_, carry):
        gc, a0, a1, a2, a3 = carry
        accs = [a0, a1, a2, a3]
        for _u in range(_UNROLL):
            for o in range(_NO):
                f = smooth2(po[o] - gc)
                t = jnp.sum(f, axis=1, keepdims=True)
                s = t[0:_RB, :] + t[_RB:, :]
                k = (_u * _NO + o) % 4
                accs[k] = jnp.minimum(accs[k], s)
            gc = jnp.roll(gc, -_RB, axis=1)
        return (gc, *accs)

    acc0 = jnp.full((_RB, 1), jnp.inf, dtype=jnp.float32)
    out = jax.lax.fori_loop(
        0, 128 // (_RB * _UNROLL), body, (g16, acc0, acc0, acc0, acc0)
    )
    acc = jnp.minimum(jnp.minimum(out[1], out[2]), jnp.minimum(out[3], out[4]))
    o_ref[0, :, :] = jnp.min(acc, axis=(0, 1), keepdims=True)


@jax.jit
def kernel(pred, gt):
    b = pred.shape[0]
    p = jnp.transpose(pred, (0, 2, 1))
    g = jnp.transpose(gt, (0, 2, 1))
    mins = pl.pallas_call(
        _poly_loss_kernel,
        grid=(b,),
        in_specs=[
            pl.BlockSpec((1, 2, _PNUM), lambda i: (i, 0, 0)),
            pl.BlockSpec((1, 2, _PNUM), lambda i: (i, 0, 0)),
        ],
        out_specs=pl.BlockSpec((1, 1, 1), lambda i: (i, 0, 0)),
        out_shape=jax.ShapeDtypeStruct((b, 1, 1), jnp.float32),
        compiler_params=pltpu.CompilerParams(
            dimension_semantics=("parallel",),
        ),
    )(p, g)
    return jnp.mean(mins) / (2.0 * _PNUM)


# speedup vs baseline: 1.1986x; 1.1986x over previous
import jax
import jax.numpy as jnp
from jax.experimental import pallas as pl
from jax.experimental.pallas import tpu as pltpu

_PNUM = 1024
_RB = 8
_NO = _PNUM // 128
_UNROLL = 16


def _poly_loss_kernel(p_ref, g_ref, o_ref):
    px = p_ref[0, 0:1, :]
    py = p_ref[0, 1:2, :]
    gx = g_ref[0, 0:1, :]
    gy = g_ref[0, 1:2, :]

    def _roll(v, r):
        return v if r == 0 else jnp.roll(v, -r, axis=1)

    gx8 = jnp.concatenate([_roll(gx, r) for r in range(_RB)], axis=0)
    gy8 = jnp.concatenate([_roll(gy, r) for r in range(_RB)], axis=0)

    pxb = jnp.broadcast_to(px, (_RB, _PNUM))
    pyb = jnp.broadcast_to(py, (_RB, _PNUM))
    pxo = [pxb] + [jnp.roll(pxb, 128 * o, axis=1) for o in range(1, _NO)]
    pyo = [pyb] + [jnp.roll(pyb, 128 * o, axis=1) for o in range(1, _NO)]

    def smooth2(d):
        a = jnp.abs(d)
        m = jnp.minimum(a, 1.0)
        return m * (a + a - m)

    def body(_, carry):
        gxc, gyc, a0, a1, a2, a3 = carry
        accs = [a0, a1, a2, a3]
        for _u in range(_UNROLL):
            for o in range(_NO):
                f = smooth2(pxo[o] - gxc) + smooth2(pyo[o] - gyc)
                s = jnp.sum(f, axis=1, keepdims=True)
                k = (_u * _NO + o) % 4
                accs[k] = jnp.minimum(accs[k], s)
            gxc = jnp.roll(gxc, -_RB, axis=1)
            gyc = jnp.roll(gyc, -_RB, axis=1)
        return (gxc, gyc, *accs)

    acc0 = jnp.full((_RB, 1), jnp.inf, dtype=jnp.float32)
    out = jax.lax.fori_loop(
        0, 128 // (_RB * _UNROLL), body, (gx8, gy8, acc0, acc0, acc0, acc0)
    )
    acc = jnp.minimum(jnp.minimum(out[2], out[3]), jnp.minimum(out[4], out[5]))
    o_ref[0, :, :] = jnp.min(acc, axis=(0, 1), keepdims=True)


@jax.jit
def kernel(pred, gt):
    b = pred.shape[0]
    p = jnp.transpose(pred, (0, 2, 1))
    g = jnp.transpose(gt, (0, 2, 1))
    mins = pl.pallas_call(
        _poly_loss_kernel,
        grid=(b,),
        in_specs=[
            pl.BlockSpec((1, 2, _PNUM), lambda i: (i, 0, 0)),
            pl.BlockSpec((1, 2, _PNUM), lambda i: (i, 0, 0)),
        ],
        out_specs=pl.BlockSpec((1, 1, 1), lambda i: (i, 0, 0)),
        out_shape=jax.ShapeDtypeStruct((b, 1, 1), jnp.float32),
        compiler_params=pltpu.CompilerParams(
            dimension_semantics=("parallel",),
        ),
    )(p, g)
    return jnp.mean(mins) / (2.0 * _PNUM)
